# baseline (speedup 1.0000x reference)
import jax
import jax.numpy as jnp
from jax import lax
from jax.experimental import pallas as pl
from jax.experimental.pallas import tpu as pltpu
from jax.experimental.pallas import tpu_sc as plsc

_NC, _NS = 2, 16
_NW = _NC * _NS
_CH = 128


def _sc_gather(uq, iq, um_i, im_i, eug4, eig4, eum, eim):
    B = uq.shape[0]
    bpw = B // _NW
    nch = bpw // _CH
    DM = eum.shape[1]
    f32 = jnp.float32
    mesh = plsc.VectorSubcoreMesh(core_axis_name="c", subcore_axis_name="s",
                                  num_cores=_NC, num_subcores=_NS)

    def body(uq_h, iq_h, umi_h, imi_h, eug4_h, eig4_h, eum_h, eim_h,
             ug4_o, ig4_o, um_o, im_o,
             idx, bufa, bufb, sema, semb):
        wid = lax.axis_index("s") * _NC + lax.axis_index("c")
        base = wid * bpw
        for k, (ih, th) in enumerate(((uq_h, eug4_h), (iq_h, eig4_h),
                                      (umi_h, eum_h), (imi_h, eim_h))):
            pltpu.sync_copy(ih.at[pl.ds(base, bpw)], idx.at[k])
        half = bpw // 2
        nh = half // _CH
        tables = ((eug4_h, ug4_o), (eig4_h, ig4_o), (eum_h, um_o),
                  (eim_h, im_o))
        stages = [(k, hf) for k, _ in enumerate(tables) for hf in range(2)]
        live = []
        for si, (k, hf) in enumerate(stages):
            th, oh = tables[k]
            buf = bufa if si % 2 == 0 else bufb
            sem = sema if si % 2 == 0 else semb
            off = hf * half
            cps = [pltpu.async_copy(
                       th.at[idx.at[k, pl.ds(off + j * _CH, _CH)]],
                       buf.at[pl.ds(j * _CH, _CH)], sem)
                   for j in range(nh)]
            live.append((cps, buf, oh, off))
            if len(live) >= 2:
                pcps, pbuf, poh, poff = live[-2]
                for c in pcps:
                    c.wait()
                pltpu.sync_copy(pbuf, poh.at[pl.ds(base + poff, half)])
        pcps, pbuf, poh, poff = live[-1]
        for c in pcps:
            c.wait()
        pltpu.sync_copy(pbuf, poh.at[pl.ds(base + poff, half)])

    k = pl.kernel(
        body,
        out_type=(jax.ShapeDtypeStruct((B, DM), f32),
                  jax.ShapeDtypeStruct((B, DM), f32),
                  jax.ShapeDtypeStruct((B, DM), f32),
                  jax.ShapeDtypeStruct((B, DM), f32)),
        mesh=mesh,
        scratch_types=[
            pltpu.VMEM((4, bpw), jnp.int32),
            pltpu.VMEM((bpw // 2, DM), f32),
            pltpu.VMEM((bpw // 2, DM), f32),
            pltpu.SemaphoreType.DMA,
            pltpu.SemaphoreType.DMA,
        ],
    )
    return k(uq, iq, um_i, im_i, eug4, eig4, eum, eim)


def _tc_mlp(ug4, ig4, ou, oi, um, im, W0a, W0b, b0, W1, b1, W2, b2,
            wpg, wph, bp):
    B, DM = um.shape
    F = wpg.shape[1]
    BT = 2048
    f32 = jnp.float32

    def body(ug4_r, ig4_r, ou_r, oi_r, um_r, im_r, W0a_r, W0b_r, b0_r,
             W1_r, b1_r, W2_r, b2_r, wpg_r, wph_r, bp_r, out_r):
        h = jnp.dot(um_r[...], W0a_r[...], preferred_element_type=f32)
        h = h + jnp.dot(im_r[...], W0b_r[...], preferred_element_type=f32)
        h = jnp.maximum(h + b0_r[...], 0.0)
        h = jnp.maximum(
            jnp.dot(h, W1_r[...], preferred_element_type=f32) + b1_r[...], 0.0)
        h = jnp.maximum(
            jnp.dot(h, W2_r[...], preferred_element_type=f32) + b2_r[...], 0.0)
        g4 = ug4_r[...]
        i4 = ig4_r[...]
        ou = ou_r[...]
        oi = oi_r[...]
        acc = jnp.sum(h * wph_r[...], axis=1, keepdims=True) + bp_r[0]
        for a in range(4):
            ga = g4[:, a * F:(a + 1) * F]
            for b in range(4):
                s = jnp.sum(ga * i4[:, b * F:(b + 1) * F] * wpg_r[...],
                            axis=1, keepdims=True)
                acc = acc + jnp.where((ou == a) & (oi == b), s, 0.0)
        out_r[...] = acc

    full = lambda shape: pl.BlockSpec(shape, lambda i: (0, 0))
    row = lambda w: pl.BlockSpec((BT, w), lambda i: (i, 0))
    out = pl.pallas_call(
        body,
        grid=(B // BT,),
        in_specs=[
            row(DM), row(DM), row(1), row(1), row(DM), row(DM),
            full((DM, DM)), full((DM, DM)), full((1, DM)),
            full((DM, DM // 2)), full((1, DM // 2)),
            full((DM // 2, DM // 4)), full((1, DM // 4)),
            full((1, F)), full((1, F)),
            pl.BlockSpec(memory_space=pltpu.SMEM),
        ],
        out_specs=pl.BlockSpec((BT, 1), lambda i: (i, 0)),
        out_shape=jax.ShapeDtypeStruct((B, 1), f32),
    )(ug4, ig4, ou, oi, um, im, W0a, W0b, b0, W1, b1, W2, b2, wpg, wph, bp)
    return out


def kernel(user, item, emb_user_gmf, emb_item_gmf, emb_user_mlp, emb_item_mlp,
           W0, b0, W1, b1, W2, b2, Wp, bp):
    F = emb_user_gmf.shape[1]
    DM = emb_user_mlp.shape[1]
    per = DM // F
    user = user.astype(jnp.int32)
    item = item.astype(jnp.int32)
    eug4 = emb_user_gmf.reshape(-1, DM)
    eig4 = emb_item_gmf.reshape(-1, DM)
    ug4, ig4, um, im = _sc_gather(user // per, item // per, user, item,
                                  eug4, eig4, emb_user_mlp, emb_item_mlp)
    pred = _tc_mlp(ug4, ig4,
                   (user % per).reshape(-1, 1), (item % per).reshape(-1, 1),
                   um, im,
                   W0[:DM], W0[DM:], b0.reshape(1, DM),
                   W1, b1.reshape(1, DM // 2),
                   W2, b2.reshape(1, DM // 4),
                   Wp[:F].reshape(1, F), Wp[F:].reshape(1, F),
                   bp)
    return pred.reshape(-1)


# speedup vs baseline: 1.7535x; 1.7535x over previous
import jax
import jax.numpy as jnp
from jax import lax
from jax.experimental import pallas as pl
from jax.experimental.pallas import tpu as pltpu
from jax.experimental.pallas import tpu_sc as plsc

_NC, _NS = 2, 16
_NW = _NC * _NS
_CH = 128
_GC = 256


def _sc_gather_mlp(user, item, eum, eim):
    B = user.shape[0]
    bpw = B // _NW
    nch = bpw // _CH
    DM = eum.shape[1]
    f32 = jnp.float32
    mesh = plsc.VectorSubcoreMesh(core_axis_name="c", subcore_axis_name="s",
                                  num_cores=_NC, num_subcores=_NS)

    def body(user_h, item_h, eum_h, eim_h, um_o, im_o, uidx, iidx, rbig,
             sem, sem2):
        wid = lax.axis_index("s") * _NC + lax.axis_index("c")
        base = wid * bpw
        pltpu.sync_copy(user_h.at[pl.ds(base, bpw)], uidx)
        pltpu.sync_copy(item_h.at[pl.ds(base, bpw)], iidx)
        cps = []
        for j in range(nch):
            sl = pl.ds(j * _CH, _CH)
            cps.append(pltpu.async_copy(eum_h.at[uidx.at[sl]], rbig.at[sl], sem))
        for c in cps:
            c.wait()
        pltpu.sync_copy(rbig, um_o.at[pl.ds(base, bpw)])
        cps2 = []
        for j in range(nch):
            sl = pl.ds(j * _CH, _CH)
            cps2.append(pltpu.async_copy(eim_h.at[iidx.at[sl]], rbig.at[sl], sem2))
        for c in cps2:
            c.wait()
        pltpu.sync_copy(rbig, im_o.at[pl.ds(base, bpw)])

    k = pl.kernel(
        body,
        out_type=(jax.ShapeDtypeStruct((B, DM), f32),
                  jax.ShapeDtypeStruct((B, DM), f32)),
        mesh=mesh,
        scratch_types=[
            pltpu.VMEM((bpw,), jnp.int32),
            pltpu.VMEM((bpw,), jnp.int32),
            pltpu.VMEM((bpw, DM), f32),
            pltpu.SemaphoreType.DMA,
            pltpu.SemaphoreType.DMA,
        ],
    )
    return k(user, item, eum, eim)


def _sc_gather_gmf(user, item, eug, eig):
    B = user.shape[0]
    bpw = B // _NW
    ngc = bpw // _GC
    F = eug.shape[1]
    f32 = jnp.float32
    mesh = plsc.VectorSubcoreMesh(core_axis_name="c", subcore_axis_name="s",
                                  num_cores=_NC, num_subcores=_NS)

    def body(user_h, item_h, eug_h, eig_h, ug_o, ig_o, uidx, iidx, rg, semg):
        wid = lax.axis_index("s") * _NC + lax.axis_index("c")
        base = wid * bpw
        pltpu.sync_copy(user_h.at[pl.ds(base, bpw)], uidx)
        pltpu.sync_copy(item_h.at[pl.ds(base, bpw)], iidx)
        for tbl_h, idx, out_o in ((eug_h, uidx, ug_o), (eig_h, iidx, ig_o)):
            for c in range(ngc):
                def gmf_group(g, carry, c=c, tbl_h=tbl_h, idx=idx):
                    vec = idx[pl.ds(c * _GC + g * 16, 16)]
                    for l in range(16):
                        pltpu.async_copy(tbl_h.at[vec[l]],
                                         rg.at[g * 16 + l], semg)
                    return carry
                lax.fori_loop(0, _GC // 16, gmf_group, 0)
                pltpu.make_async_copy(tbl_h.at[pl.ds(0, _GC)], rg, semg).wait()
                pltpu.sync_copy(rg, out_o.at[pl.ds(base + c * _GC, _GC)])

    k = pl.kernel(
        body,
        out_type=(jax.ShapeDtypeStruct((B, F), f32),
                  jax.ShapeDtypeStruct((B, F), f32)),
        mesh=mesh,
        scratch_types=[
            pltpu.VMEM((bpw,), jnp.int32),
            pltpu.VMEM((bpw,), jnp.int32),
            pltpu.VMEM((_GC, F), f32),
            pltpu.SemaphoreType.DMA,
        ],
    )
    return k(user, item, eug, eig)


def _tc_mlp(ug, ig, um, im, W0a, W0b, b0, W1, b1, W2, b2, wpg, wph, bp):
    B, F = ug.shape
    DM = um.shape[1]
    BT = 2048
    f32 = jnp.float32

    def body(ug_r, ig_r, um_r, im_r, W0a_r, W0b_r, b0_r, W1_r, b1_r,
             W2_r, b2_r, wpg_r, wph_r, bp_r, out_r):
        h = jnp.dot(um_r[...], W0a_r[...], preferred_element_type=f32)
        h = h + jnp.dot(im_r[...], W0b_r[...], preferred_element_type=f32)
        h = jnp.maximum(h + b0_r[...], 0.0)
        h = jnp.maximum(
            jnp.dot(h, W1_r[...], preferred_element_type=f32) + b1_r[...], 0.0)
        h = jnp.maximum(
            jnp.dot(h, W2_r[...], preferred_element_type=f32) + b2_r[...], 0.0)
        g = ug_r[...] * ig_r[...]
        p = (jnp.sum(g * wpg_r[...], axis=1, keepdims=True)
             + jnp.sum(h * wph_r[...], axis=1, keepdims=True) + bp_r[0])
        out_r[...] = p

    full = lambda shape: pl.BlockSpec(shape, lambda i: (0, 0))
    out = pl.pallas_call(
        body,
        grid=(B // BT,),
        in_specs=[
            pl.BlockSpec((BT, F), lambda i: (i, 0)),
            pl.BlockSpec((BT, F), lambda i: (i, 0)),
            pl.BlockSpec((BT, DM), lambda i: (i, 0)),
            pl.BlockSpec((BT, DM), lambda i: (i, 0)),
            full((DM, DM)), full((DM, DM)), full((1, DM)),
            full((DM, DM // 2)), full((1, DM // 2)),
            full((DM // 2, DM // 4)), full((1, DM // 4)),
            full((1, F)), full((1, F)),
            pl.BlockSpec(memory_space=pltpu.SMEM),
        ],
        out_specs=pl.BlockSpec((BT, 1), lambda i: (i, 0)),
        out_shape=jax.ShapeDtypeStruct((B, 1), f32),
    )(ug, ig, um, im, W0a, W0b, b0, W1, b1, W2, b2, wpg, wph, bp)
    return out


def kernel(user, item, emb_user_gmf, emb_item_gmf, emb_user_mlp, emb_item_mlp,
           W0, b0, W1, b1, W2, b2, Wp, bp):
    F = emb_user_gmf.shape[1]
    DM = emb_user_mlp.shape[1]
    user = user.astype(jnp.int32)
    item = item.astype(jnp.int32)
    um, im = _sc_gather_mlp(user, item, emb_user_mlp, emb_item_mlp)
    ug, ig = _sc_gather_gmf(user, item, emb_user_gmf, emb_item_gmf)
    pred = _tc_mlp(ug, ig, um, im,
                   W0[:DM], W0[DM:], b0.reshape(1, DM),
                   W1, b1.reshape(1, DM // 2),
                   W2, b2.reshape(1, DM // 4),
                   Wp[:F].reshape(1, F), Wp[F:].reshape(1, F),
                   bp)
    return pred.reshape(-1)


# speedup vs baseline: 2.7076x; 1.5441x over previous
import jax
import jax.numpy as jnp
from jax import lax
from jax.experimental import pallas as pl
from jax.experimental.pallas import tpu as pltpu
from jax.experimental.pallas import tpu_sc as plsc

_NC, _NS = 2, 16
_NW = _NC * _NS
_CH = 128
_BC = 4096


def _sc_gather_mlp(user, item, eum, eim):
    B = user.shape[0]
    bpw = B // _NW
    nch = bpw // _CH
    DM = eum.shape[1]
    f32 = jnp.float32
    mesh = plsc.VectorSubcoreMesh(core_axis_name="c", subcore_axis_name="s",
                                  num_cores=_NC, num_subcores=_NS)

    def body(user_h, item_h, eum_h, eim_h, um_o, im_o, uidx, iidx, rbig,
             sem, sem2):
        wid = lax.axis_index("s") * _NC + lax.axis_index("c")
        base = wid * bpw
        pltpu.sync_copy(user_h.at[pl.ds(base, bpw)], uidx)
        pltpu.sync_copy(item_h.at[pl.ds(base, bpw)], iidx)
        cps = []
        for j in range(nch):
            sl = pl.ds(j * _CH, _CH)
            cps.append(pltpu.async_copy(eum_h.at[uidx.at[sl]], rbig.at[sl], sem))
        for c in cps:
            c.wait()
        pltpu.sync_copy(rbig, um_o.at[pl.ds(base, bpw)])
        cps2 = []
        for j in range(nch):
            sl = pl.ds(j * _CH, _CH)
            cps2.append(pltpu.async_copy(eim_h.at[iidx.at[sl]], rbig.at[sl], sem2))
        for c in cps2:
            c.wait()
        pltpu.sync_copy(rbig, im_o.at[pl.ds(base, bpw)])

    k = pl.kernel(
        body,
        out_type=(jax.ShapeDtypeStruct((B, DM), f32),
                  jax.ShapeDtypeStruct((B, DM), f32)),
        mesh=mesh,
        scratch_types=[
            pltpu.VMEM((bpw,), jnp.int32),
            pltpu.VMEM((bpw,), jnp.int32),
            pltpu.VMEM((bpw, DM), f32),
            pltpu.SemaphoreType.DMA,
            pltpu.SemaphoreType.DMA,
        ],
    )
    return k(user, item, eum, eim)


def _sc_gmf_prod(user, item, eugT, eigT):
    B = user.shape[0]
    F, V = eugT.shape
    nbc = B // _BC
    f32 = jnp.float32
    mesh = plsc.VectorSubcoreMesh(core_axis_name="c", subcore_axis_name="s",
                                  num_cores=_NC, num_subcores=_NS)

    def body(user_h, item_h, eugT_h, eigT_h, prod_o, feat, idxc, vals):
        wid = lax.axis_index("s") * _NC + lax.axis_index("c")
        pltpu.sync_copy(eugT_h.at[wid], feat)
        for c in range(nbc):
            pltpu.sync_copy(user_h.at[pl.ds(c * _BC, _BC)], idxc)
            def ga(g, carry, c=c):
                vec = idxc[pl.ds(g * 16, 16)]
                vals[pl.ds(c * _BC + g * 16, 16)] = plsc.load_gather(
                    feat, [vec])
                return carry
            lax.fori_loop(0, _BC // 16, ga, 0)
        pltpu.sync_copy(eigT_h.at[wid], feat)
        for c in range(nbc):
            pltpu.sync_copy(item_h.at[pl.ds(c * _BC, _BC)], idxc)
            def gb(g, carry, c=c):
                vec = idxc[pl.ds(g * 16, 16)]
                sl = pl.ds(c * _BC + g * 16, 16)
                vals[sl] = vals[sl] * plsc.load_gather(feat, [vec])
                return carry
            lax.fori_loop(0, _BC // 16, gb, 0)
        pltpu.sync_copy(vals, prod_o.at[wid])

    k = pl.kernel(
        body,
        out_type=jax.ShapeDtypeStruct((F, B), f32),
        mesh=mesh,
        compiler_params=pltpu.CompilerParams(needs_layout_passes=False),
        scratch_types=[
            pltpu.VMEM((V,), f32),
            pltpu.VMEM((_BC,), jnp.int32),
            pltpu.VMEM((B,), f32),
        ],
    )
    return k(user, item, eugT, eigT)


def _tc_mlp(prod, um, im, W0a, W0b, b0, W1, b1, W2, b2, wpgT, wph, bp):
    F, B = prod.shape
    DM = um.shape[1]
    BT = 2048
    f32 = jnp.float32

    def body(prod_r, um_r, im_r, W0a_r, W0b_r, b0_r, W1_r, b1_r,
             W2_r, b2_r, wpgT_r, wph_r, bp_r, out_r):
        h = jnp.dot(um_r[...], W0a_r[...], preferred_element_type=f32)
        h = h + jnp.dot(im_r[...], W0b_r[...], preferred_element_type=f32)
        h = jnp.maximum(h + b0_r[...], 0.0)
        h = jnp.maximum(
            jnp.dot(h, W1_r[...], preferred_element_type=f32) + b1_r[...], 0.0)
        h = jnp.maximum(
            jnp.dot(h, W2_r[...], preferred_element_type=f32) + b2_r[...], 0.0)
        p = (jnp.sum(prod_r[...] * wpgT_r[...], axis=0)
             + jnp.sum(h * wph_r[...], axis=1) + bp_r[0])
        out_r[...] = p

    full = lambda shape: pl.BlockSpec(shape, lambda i: tuple(0 for _ in shape))
    out = pl.pallas_call(
        body,
        grid=(B // BT,),
        in_specs=[
            pl.BlockSpec((F, BT), lambda i: (0, i)),
            pl.BlockSpec((BT, DM), lambda i: (i, 0)),
            pl.BlockSpec((BT, DM), lambda i: (i, 0)),
            full((DM, DM)), full((DM, DM)), full((1, DM)),
            full((DM, DM // 2)), full((1, DM // 2)),
            full((DM // 2, DM // 4)), full((1, DM // 4)),
            full((F, 1)), full((1, F)),
            pl.BlockSpec(memory_space=pltpu.SMEM),
        ],
        out_specs=pl.BlockSpec((BT,), lambda i: (i,)),
        out_shape=jax.ShapeDtypeStruct((B,), f32),
    )(prod, um, im, W0a, W0b, b0, W1, b1, W2, b2, wpgT, wph, bp)
    return out


def kernel(user, item, emb_user_gmf, emb_item_gmf, emb_user_mlp, emb_item_mlp,
           W0, b0, W1, b1, W2, b2, Wp, bp):
    F = emb_user_gmf.shape[1]
    DM = emb_user_mlp.shape[1]
    user = user.astype(jnp.int32)
    item = item.astype(jnp.int32)
    um, im = _sc_gather_mlp(user, item, emb_user_mlp, emb_item_mlp)
    prod = _sc_gmf_prod(user, item, emb_user_gmf.T, emb_item_gmf.T)
    return _tc_mlp(prod, um, im,
                   W0[:DM], W0[DM:], b0.reshape(1, DM),
                   W1, b1.reshape(1, DM // 2),
                   W2, b2.reshape(1, DM // 4),
                   Wp[:F].reshape(F, 1), Wp[F:].reshape(1, F),
                   bp)


# speedup vs baseline: 2.8651x; 1.0582x over previous
import jax
import jax.numpy as jnp
from jax import lax
from jax.experimental import pallas as pl
from jax.experimental.pallas import tpu as pltpu
from jax.experimental.pallas import tpu_sc as plsc

_NC, _NS = 2, 16
_NW = _NC * _NS
_CH = 128
_BC = 4096


def _sc_gather(user, item, eugT, eigT, eum, eim):
    B = user.shape[0]
    bpw = B // _NW
    nch = bpw // _CH
    nbc = B // _BC
    F, V = eugT.shape
    DM = eum.shape[1]
    f32 = jnp.float32
    mesh = plsc.VectorSubcoreMesh(core_axis_name="c", subcore_axis_name="s",
                                  num_cores=_NC, num_subcores=_NS)

    def body(user_h, item_h, eugT_h, eigT_h, eum_h, eim_h,
             prod_o, um_o, im_o,
             feat, valsc, idxc, mbuf, uidx, iidx, sems, semg):
        wid = lax.axis_index("s") * _NC + lax.axis_index("c")
        base = wid * bpw
        pltpu.sync_copy(user_h.at[pl.ds(base, bpw)], uidx)
        pltpu.sync_copy(item_h.at[pl.ds(base, bpw)], iidx)

        mlp = ((eum_h, uidx, um_o), (eim_h, iidx, im_o))
        nst = 2 * nch

        def fire(j):
            th, ix, oo = mlp[j // nch]
            jj = j % nch
            cp = pltpu.async_copy(th.at[ix.at[pl.ds(jj * _CH, _CH)]],
                                  mbuf, sems)
            return (cp, oo, jj)

        def drain(cur, j):
            cp, oo, jj = cur
            cp.wait()
            pltpu.sync_copy(mbuf, oo.at[pl.ds(base + jj * _CH, _CH)])
            return fire(j + 1) if j + 1 < nst else None

        cur = fire(0)
        step = 0
        pltpu.sync_copy(eugT_h.at[wid], feat)
        for c in range(nbc):
            pltpu.sync_copy(user_h.at[pl.ds(c * _BC, _BC)], idxc)
            def ga(g, carry):
                for u in range(4):
                    vec = idxc[pl.ds((g * 4 + u) * 16, 16)]
                    valsc[pl.ds((g * 4 + u) * 16, 16)] = plsc.load_gather(
                        feat, [vec])
                return carry
            lax.fori_loop(0, _BC // 64, ga, 0)
            pltpu.sync_copy(valsc, prod_o.at[wid, pl.ds(c * _BC, _BC)])
            cur = drain(cur, step)
            step += 1
        pltpu.sync_copy(eigT_h.at[wid], feat)
        for c in range(nbc):
            pltpu.sync_copy(item_h.at[pl.ds(c * _BC, _BC)], idxc)
            pltpu.sync_copy(prod_o.at[wid, pl.ds(c * _BC, _BC)], valsc)
            def gb(g, carry):
                for u in range(4):
                    sl = pl.ds((g * 4 + u) * 16, 16)
                    valsc[sl] = valsc[sl] * plsc.load_gather(
                        feat, [idxc[sl]])
                return carry
            lax.fori_loop(0, _BC // 64, gb, 0)
            pltpu.sync_copy(valsc, prod_o.at[wid, pl.ds(c * _BC, _BC)])
            if cur is not None:
                cur = drain(cur, step)
                step += 1

    k = pl.kernel(
        body,
        out_type=(jax.ShapeDtypeStruct((F, B), f32),
                  jax.ShapeDtypeStruct((B, DM), f32),
                  jax.ShapeDtypeStruct((B, DM), f32)),
        mesh=mesh,
        compiler_params=pltpu.CompilerParams(needs_layout_passes=False),
        scratch_types=[
            pltpu.VMEM((V,), f32),
            pltpu.VMEM((_BC,), f32),
            pltpu.VMEM((_BC,), jnp.int32),
            pltpu.VMEM((_CH, DM), f32),
            pltpu.VMEM((bpw,), jnp.int32),
            pltpu.VMEM((bpw,), jnp.int32),
            pltpu.SemaphoreType.DMA,
            pltpu.SemaphoreType.DMA,
        ],
    )
    return k(user, item, eugT, eigT, eum, eim)


def _tc_mlp(prod, um, im, W0a, W0b, b0, W1, b1, W2, b2, wpgT, wph, bp):
    F, B = prod.shape
    DM = um.shape[1]
    BT = 2048
    f32 = jnp.float32

    def body(prod_r, um_r, im_r, W0a_r, W0b_r, b0_r, W1_r, b1_r,
             W2_r, b2_r, wpgT_r, wph_r, bp_r, out_r):
        h = jnp.dot(um_r[...], W0a_r[...], preferred_element_type=f32)
        h = h + jnp.dot(im_r[...], W0b_r[...], preferred_element_type=f32)
        h = jnp.maximum(h + b0_r[...], 0.0)
        h = jnp.maximum(
            jnp.dot(h, W1_r[...], preferred_element_type=f32) + b1_r[...], 0.0)
        h = jnp.maximum(
            jnp.dot(h, W2_r[...], preferred_element_type=f32) + b2_r[...], 0.0)
        p = (jnp.sum(prod_r[...] * wpgT_r[...], axis=0)
             + jnp.sum(h * wph_r[...], axis=1) + bp_r[0])
        out_r[...] = p

    full = lambda shape: pl.BlockSpec(shape, lambda i: tuple(0 for _ in shape))
    out = pl.pallas_call(
        body,
        grid=(B // BT,),
        in_specs=[
            pl.BlockSpec((F, BT), lambda i: (0, i)),
            pl.BlockSpec((BT, DM), lambda i: (i, 0)),
            pl.BlockSpec((BT, DM), lambda i: (i, 0)),
            full((DM, DM)), full((DM, DM)), full((1, DM)),
            full((DM, DM // 2)), full((1, DM // 2)),
            full((DM // 2, DM // 4)), full((1, DM // 4)),
            full((F, 1)), full((1, F)),
            pl.BlockSpec(memory_space=pltpu.SMEM),
        ],
        out_specs=pl.BlockSpec((BT,), lambda i: (i,)),
        out_shape=jax.ShapeDtypeStruct((B,), f32),
    )(prod, um, im, W0a, W0b, b0, W1, b1, W2, b2, wpgT, wph, bp)
    return out


def kernel(user, item, emb_user_gmf, emb_item_gmf, emb_user_mlp, emb_item_mlp,
           W0, b0, W1, b1, W2, b2, Wp, bp):
    F = emb_user_gmf.shape[1]
    DM = emb_user_mlp.shape[1]
    user = user.astype(jnp.int32)
    item = item.astype(jnp.int32)
    prod, um, im = _sc_gather(user, item, emb_user_gmf.T, emb_item_gmf.T,
                              emb_user_mlp, emb_item_mlp)
    return _tc_mlp(prod, um, im,
                   W0[:DM], W0[DM:], b0.reshape(1, DM),
                   W1, b1.reshape(1, DM // 2),
                   W2, b2.reshape(1, DM // 4),
                   Wp[:F].reshape(F, 1), Wp[F:].reshape(1, F),
                   bp)
